# Initial kernel scaffold; baseline (speedup 1.0000x reference)
import jax
import jax.numpy as jnp
from jax.experimental import pallas as pl


def kernel(x, table):
    raise NotImplementedError("write your pallas kernel here")


# speedup vs baseline: 3.6077x; 3.6077x over previous
import functools

import jax
import jax.numpy as jnp
from jax import lax
from jax.experimental import pallas as pl
from jax.experimental.pallas import tpu as pltpu
from jax.experimental.pallas import tpu_sc as plsc

_C = 128


@functools.partial(jax.jit, static_argnums=(2, 3, 4, 5))
def _embed(idx, table, n_chunks, chunk, nc, ns):
    nw = nc * ns
    d = table.shape[1]
    n = nw * n_chunks * chunk
    mesh = plsc.VectorSubcoreMesh(core_axis_name="c", subcore_axis_name="s")

    @functools.partial(
        pl.kernel,
        mesh=mesh,
        out_type=jax.ShapeDtypeStruct((n, d), table.dtype),
        compiler_params=pltpu.CompilerParams(use_tc_tiling_on_sc=False),
        scratch_types=[
            pltpu.VMEM((n_chunks, chunk), jnp.int32),
            pltpu.VMEM((chunk, d), jnp.float32),
            pltpu.SemaphoreType.DMA,
        ],
    )
    def emb(idx_hbm, table_hbm, out_hbm, idx_v, rows_v, gsem):
        wid = lax.axis_index("s") * nc + lax.axis_index("c")
        base = wid * (n_chunks * chunk)
        pltpu.sync_copy(idx_hbm.at[wid], idx_v)

        def body(g, carry):
            pltpu.async_copy(table_hbm.at[idx_v.at[g]], rows_v, gsem).wait()
            pltpu.sync_copy(rows_v, out_hbm.at[pl.ds(base + g * chunk, chunk)])
            return carry

        lax.fori_loop(0, n_chunks, body, 0)

    return emb(idx, table)


def kernel(x, table):
    b, l = x.shape
    d = table.shape[1]
    n = b * l
    info = plsc.get_sparse_core_info()
    nc, ns = info.num_cores, info.num_subcores
    nw = nc * ns
    n_chunks = n // (nw * _C)
    assert n == nw * n_chunks * _C
    idx = x.reshape(nw, n_chunks, _C).astype(jnp.int32)
    out = _embed(idx, table, n_chunks, _C, nc, ns)
    return out.reshape(b, l, d)


# speedup vs baseline: 4.1952x; 1.1628x over previous
import functools

import jax
import jax.numpy as jnp
from jax import lax
from jax.experimental import pallas as pl
from jax.experimental.pallas import tpu as pltpu
from jax.experimental.pallas import tpu_sc as plsc

_C = 128
_K = 5


@functools.partial(jax.jit, static_argnums=(2, 3, 4, 5, 6))
def _embed(idx, table, n_super, k, chunk, nc, ns):
    nw = nc * ns
    d = table.shape[1]
    sc_rows = k * chunk
    n = nw * n_super * sc_rows
    n_pairs = n_super // 2
    mesh = plsc.VectorSubcoreMesh(core_axis_name="c", subcore_axis_name="s")

    @functools.partial(
        pl.kernel,
        mesh=mesh,
        out_type=jax.ShapeDtypeStruct((n, d), table.dtype),
        compiler_params=pltpu.CompilerParams(use_tc_tiling_on_sc=False),
        scratch_types=[
            pltpu.VMEM((n_super * k, chunk), jnp.int32),
            pltpu.VMEM((sc_rows, d), jnp.float32),
            pltpu.VMEM((sc_rows, d), jnp.float32),
            pltpu.SemaphoreType.DMA,
            pltpu.SemaphoreType.DMA,
            pltpu.SemaphoreType.DMA,
            pltpu.SemaphoreType.DMA,
        ],
    )
    def emb(idx_hbm, table_hbm, out_hbm, idx_v, rows0, rows1,
            gsem0, gsem1, wsem0, wsem1):
        wid = lax.axis_index("s") * nc + lax.axis_index("c")
        base = wid * (n_super * sc_rows)
        pltpu.sync_copy(idx_hbm.at[wid], idx_v)

        def fire_gathers(sg, rows, gsem):
            for j in range(k):
                pltpu.async_copy(table_hbm.at[idx_v.at[sg * k + j]],
                                 rows.at[pl.ds(j * chunk, chunk)], gsem)

        def drain_gathers(rows, gsem):
            pltpu.make_async_copy(table_hbm.at[pl.ds(0, sc_rows)], rows,
                                  gsem).wait()

        def fire_write(sg, rows, wsem):
            pltpu.async_copy(rows, out_hbm.at[pl.ds(base + sg * sc_rows,
                                                    sc_rows)], wsem)

        def drain_write(rows, wsem):
            pltpu.make_async_copy(rows, out_hbm.at[pl.ds(base, sc_rows)],
                                  wsem).wait()

        fire_gathers(0, rows0, gsem0)

        def body(t, carry):
            @pl.when(t > 0)
            def _():
                drain_write(rows1, wsem1)

            fire_gathers(2 * t + 1, rows1, gsem1)
            drain_gathers(rows0, gsem0)
            fire_write(2 * t, rows0, wsem0)
            drain_write(rows0, wsem0)

            @pl.when(t < n_pairs - 1)
            def _():
                fire_gathers(2 * t + 2, rows0, gsem0)

            drain_gathers(rows1, gsem1)
            fire_write(2 * t + 1, rows1, wsem1)
            return carry

        lax.fori_loop(0, n_pairs, body, 0)
        drain_write(rows1, wsem1)

    return emb(idx, table)


def kernel(x, table):
    b, l = x.shape
    d = table.shape[1]
    n = b * l
    info = plsc.get_sparse_core_info()
    nc, ns = info.num_cores, info.num_subcores
    nw = nc * ns
    n_super = n // (nw * _K * _C)
    assert n == nw * n_super * _K * _C and n_super % 2 == 0
    idx = x.reshape(nw, n_super * _K, _C).astype(jnp.int32)
    out = _embed(idx, table, n_super, _K, _C, nc, ns)
    return out.reshape(b, l, d)
